# baseline (speedup 1.0000x reference)
import jax
import jax.numpy as jnp
from jax.experimental import pallas as pl
from jax.experimental.pallas import tpu as pltpu

_K = 1024
_D = 64
_BETA = 0.25
_NB = 8
_N = 1024
_BS = 2
_LSCALE = (1.0 + _BETA) / (_NB * _N * _D)


def _vq_body(z_ref, e_ref, zq_ref, idx_ref, loss_ref, iota_ref):
    @pl.when(pl.program_id(0) == 0)
    def _():
        loss_ref[0, 0] = 0.0
        iota_ref[...] = jax.lax.broadcasted_iota(
            jnp.int32, (_K, _N), 0).astype(jnp.float32)

    e = e_ref[...]
    e2 = e + e
    en = jnp.sum(e * e, axis=1, keepdims=True)
    iota = iota_ref[...]
    for s in range(_BS):
        zb = z_ref[s]
        zn = jnp.sum(zb * zb, axis=0, keepdims=True)
        ze2 = jnp.dot(e2, zb, preferred_element_type=jnp.float32)
        dis = (zn - ze2) + en
        minv = jnp.min(dis, axis=0, keepdims=True)
        idx_f = jnp.min(jnp.where(dis == minv, iota, float(_K)),
                        axis=0, keepdims=True)
        onehot = (iota == idx_f).astype(jnp.float32)
        zq = jax.lax.dot_general(e, onehot, (((0,), (0,)), ((), ())),
                                 preferred_element_type=jnp.float32)
        zq_ref[s] = zb + (zq - zb)
        idx_ref[s] = idx_f.astype(jnp.int32)
        loss_ref[0, 0] += _LSCALE * jnp.sum(minv)


def kernel(z, embs):
    z3 = z.reshape(_NB, _D, _N)
    zq3, idx3, loss = pl.pallas_call(
        _vq_body,
        grid=(_NB // _BS,),
        in_specs=[
            pl.BlockSpec((_BS, _D, _N), lambda b: (b, 0, 0)),
            pl.BlockSpec((_K, _D), lambda b: (0, 0)),
        ],
        out_specs=[
            pl.BlockSpec((_BS, _D, _N), lambda b: (b, 0, 0)),
            pl.BlockSpec((_BS, 1, _N), lambda b: (b, 0, 0)),
            pl.BlockSpec((1, 1), lambda b: (0, 0),
                         memory_space=pltpu.SMEM),
        ],
        out_shape=[
            jax.ShapeDtypeStruct((_NB, _D, _N), jnp.float32),
            jax.ShapeDtypeStruct((_NB, 1, _N), jnp.int32),
            jax.ShapeDtypeStruct((1, 1), jnp.float32),
        ],
        scratch_shapes=[pltpu.VMEM((_K, _N), jnp.float32)],
        compiler_params=pltpu.CompilerParams(
            dimension_semantics=("arbitrary",)),
    )(z3, embs)
    z_q_out = zq3.reshape(_NB, _D, 32, 32)
    min_idxs = idx3.reshape(-1)
    return (z_q_out, min_idxs, loss.reshape(()))
